# Initial kernel scaffold; baseline (speedup 1.0000x reference)
import jax
import jax.numpy as jnp
from jax.experimental import pallas as pl


def kernel(dense_features, sparse_indices, params):
    raise NotImplementedError("write your pallas kernel here")


# speedup vs baseline: 1.2245x; 1.2245x over previous
import functools

import jax
import jax.numpy as jnp
from jax.experimental import pallas as pl
from jax.experimental.pallas import tpu as pltpu
from jax.experimental.pallas import tpu_sc as plsc

B = 4096
NSPARSE = 26
EMB = 128
D0 = NSPARSE * EMB + EMB
RANK = 512

_GATHER_WINDOW = 256


def _sc_gather(table, flat_idx):
    n = flat_idx.shape[0]
    idx2 = flat_idx.reshape(1, n)
    mesh = plsc.VectorSubcoreMesh(core_axis_name="core", subcore_axis_name="subcore")

    @functools.partial(
        pl.kernel,
        out_type=jax.ShapeDtypeStruct((n, EMB), jnp.float32),
        mesh=mesh,
    )
    def k(table_hbm, idx_hbm, out_hbm):
        def body(i_vmem, o_vmem):
            pltpu.sync_copy(table_hbm.at[i_vmem.at[0]], o_vmem)

        pltpu.emit_pipeline(
            body,
            grid=(n // _GATHER_WINDOW,),
            in_specs=[pl.BlockSpec((1, _GATHER_WINDOW), index_map=lambda i: (0, i))],
            out_specs=[pl.BlockSpec((_GATHER_WINDOW, EMB), index_map=lambda i: (i, 0))],
            core_axis_name=("core", "subcore"),
            dimension_semantics=(pltpu.PARALLEL,),
        )(idx_hbm, out_hbm)

    return k(table, idx2)


def _bottom_mlp(df, w0, b0, w1, b1, w2, b2):
    def body(df_ref, w0r, b0r, w1r, b1r, w2r, b2r, out_ref):
        h = jnp.dot(df_ref[...], w0r[...], preferred_element_type=jnp.float32)
        h = jnp.maximum(h + b0r[...], 0.0)
        h = jnp.dot(h, w1r[...], preferred_element_type=jnp.float32)
        h = jnp.maximum(h + b1r[...], 0.0)
        h = jnp.dot(h, w2r[...], preferred_element_type=jnp.float32)
        out_ref[...] = jnp.maximum(h + b2r[...], 0.0)

    return pl.pallas_call(
        body,
        out_shape=jax.ShapeDtypeStruct((B, EMB), jnp.float32),
    )(df, w0, b0, w1, b1, w2, b2)


def _dcn_layer(x0, x, v, u, c, bb=512):
    def body(x0_ref, x_ref, vr, ur, cr, out_ref):
        xv = jnp.dot(x_ref[...], vr[...], preferred_element_type=jnp.float32)
        xu = jnp.dot(xv, ur[...], preferred_element_type=jnp.float32) + cr[...]
        out_ref[...] = x0_ref[...] * xu + x_ref[...]

    return pl.pallas_call(
        body,
        grid=(B // bb,),
        in_specs=[
            pl.BlockSpec((bb, D0), lambda i: (i, 0)),
            pl.BlockSpec((bb, D0), lambda i: (i, 0)),
            pl.BlockSpec((D0, RANK), lambda i: (0, 0)),
            pl.BlockSpec((RANK, D0), lambda i: (0, 0)),
            pl.BlockSpec((1, D0), lambda i: (0, 0)),
        ],
        out_specs=pl.BlockSpec((bb, D0), lambda i: (i, 0)),
        out_shape=jax.ShapeDtypeStruct((B, D0), jnp.float32),
    )(x0, x, v, u, c)


def _top_mlp(bottom, x, w0a, w0b, b0, w1, b1, w2, b2, w3, b3, w4, b4, bb=512):
    def body(bot_ref, x_ref, w0ar, w0br, b0r, w1r, b1r, w2r, b2r, w3r, b3r,
             w4r, b4r, out_ref):
        t = (jnp.dot(bot_ref[...], w0ar[...], preferred_element_type=jnp.float32)
             + jnp.dot(x_ref[...], w0br[...], preferred_element_type=jnp.float32))
        t = jnp.maximum(t + b0r[...], 0.0)
        t = jnp.maximum(jnp.dot(t, w1r[...], preferred_element_type=jnp.float32) + b1r[...], 0.0)
        t = jnp.maximum(jnp.dot(t, w2r[...], preferred_element_type=jnp.float32) + b2r[...], 0.0)
        t = jnp.maximum(jnp.dot(t, w3r[...], preferred_element_type=jnp.float32) + b3r[...], 0.0)
        logit = jnp.dot(t, w4r[...], preferred_element_type=jnp.float32) + b4r[...]
        out_ref[...] = jax.nn.sigmoid(logit)

    const = lambda i: (0, 0)
    return pl.pallas_call(
        body,
        grid=(B // bb,),
        in_specs=[
            pl.BlockSpec((bb, EMB), lambda i: (i, 0)),
            pl.BlockSpec((bb, D0), lambda i: (i, 0)),
            pl.BlockSpec(w0a.shape, const),
            pl.BlockSpec(w0b.shape, const),
            pl.BlockSpec(b0.shape, const),
            pl.BlockSpec(w1.shape, const),
            pl.BlockSpec(b1.shape, const),
            pl.BlockSpec(w2.shape, const),
            pl.BlockSpec(b2.shape, const),
            pl.BlockSpec(w3.shape, const),
            pl.BlockSpec(b3.shape, const),
            pl.BlockSpec(w4.shape, const),
            pl.BlockSpec(b4.shape, const),
        ],
        out_specs=pl.BlockSpec((bb, 1), lambda i: (i, 0)),
        out_shape=jax.ShapeDtypeStruct((B, 1), jnp.float32),
    )(bottom, x, w0a, w0b, b0, w1, b1, w2, b2, w3, b3, w4, b4)


def kernel(dense_features, sparse_indices, params):
    p = params
    idx = sparse_indices.reshape(-1).astype(jnp.int32)

    rows = _sc_gather(p["table"], idx)
    emb_flat = rows.reshape(B, NSPARSE * EMB)

    r2 = lambda a: a.reshape(1, -1)
    bottom = _bottom_mlp(
        dense_features,
        p["Wb0"], r2(p["bb0"]), p["Wb1"], r2(p["bb1"]), p["Wb2"], r2(p["bb2"]),
    )

    x0 = jnp.concatenate([emb_flat, bottom], axis=1)
    x = x0
    for i in range(3):
        x = _dcn_layer(x0, x, p[f"V{i}"], p[f"U{i}"], r2(p[f"c{i}"]))

    out = _top_mlp(
        bottom, x,
        p["Wt0"][:EMB], p["Wt0"][EMB:], r2(p["bt0"]),
        p["Wt1"], r2(p["bt1"]),
        p["Wt2"], r2(p["bt2"]),
        p["Wt3"], r2(p["bt3"]),
        p["Wt4"], r2(p["bt4"]),
    )
    return out
